# Initial kernel scaffold; baseline (speedup 1.0000x reference)
import jax
import jax.numpy as jnp
from jax.experimental import pallas as pl


def kernel(hidden_states, routing_weights, W1, W2, selected_experts):
    raise NotImplementedError("write your pallas kernel here")


# speedup vs baseline: 1.6832x; 1.6832x over previous
import functools

import jax
import jax.numpy as jnp
from jax import lax
from jax.experimental import pallas as pl
from jax.experimental.pallas import tpu as pltpu
from jax.experimental.pallas import tpu_sc as plsc

H = 768
F = 1024
NE = 64
NT = 2048
NK = 2
BT = 128
G = 96
R_PAD = G * BT

NC, NS = 2, 16
NW = NC * NS
GPW = R_PAD // NW
GCH = 64
TPW = NT // NW

@functools.lru_cache(maxsize=1)
def _sc_kernels():
    mesh = plsc.VectorSubcoreMesh(core_axis_name="c", subcore_axis_name="s")

    @functools.partial(
        pl.kernel,
        out_type=jax.ShapeDtypeStruct((R_PAD, H), jnp.float32),
        mesh=mesh,
        scratch_types=[
            pltpu.VMEM((GCH,), jnp.int32),
            pltpu.VMEM((GCH, H), jnp.float32),
            pltpu.SemaphoreType.DMA,
        ],
    )
    def dispatch(src_hbm, hs_hbm, xpad_hbm, idx_v, rows_v, sem):
        wid = lax.axis_index("s") * NC + lax.axis_index("c")
        for c in range(GPW // GCH):
            start = wid * GPW + c * GCH
            pltpu.sync_copy(src_hbm.at[pl.ds(start, GCH)], idx_v)
            pltpu.async_copy(hs_hbm.at[idx_v], rows_v, sem).wait()
            pltpu.sync_copy(rows_v, xpad_hbm.at[pl.ds(start, GCH)])

    @functools.partial(
        pl.kernel,
        out_type=jax.ShapeDtypeStruct((NT, H), jnp.float32),
        mesh=mesh,
        scratch_types=[
            pltpu.VMEM((TPW,), jnp.int32),
            pltpu.VMEM((TPW,), jnp.int32),
            pltpu.VMEM((TPW, H), jnp.float32),
            pltpu.VMEM((TPW, H), jnp.float32),
            pltpu.SemaphoreType.DMA,
        ],
    )
    def combine(idx0_hbm, idx1_hbm, opad_hbm, fin_hbm, i0_v, i1_v, r0_v, r1_v, sem):
        wid = lax.axis_index("s") * NC + lax.axis_index("c")
        base = wid * TPW
        pltpu.sync_copy(idx0_hbm.at[pl.ds(base, TPW)], i0_v)
        pltpu.sync_copy(idx1_hbm.at[pl.ds(base, TPW)], i1_v)
        cp0 = pltpu.async_copy(opad_hbm.at[i0_v], r0_v, sem)
        cp1 = pltpu.async_copy(opad_hbm.at[i1_v], r1_v, sem)
        cp0.wait()
        cp1.wait()

        def add_row(t, carry):
            for j in range(H // 16):
                sl = pl.ds(j * 16, 16)
                r0_v[t, sl] = r0_v[t, sl] + r1_v[t, sl]
            return carry

        lax.fori_loop(0, TPW, add_row, 0)
        pltpu.sync_copy(r0_v, fin_hbm.at[pl.ds(base, TPW)])

    return dispatch, combine


def _ffn_body(te_ref, nt_ref, x_ref, w1_ref, w2_ref, wr_ref, o_ref):
    g = pl.program_id(0)

    @pl.when(g < nt_ref[0])
    def _():
        x = x_ref[...]
        h = jnp.dot(x, w1_ref[0], preferred_element_type=jnp.float32)
        a = h[:, :F]
        b = h[:, F:]
        act = (a * jax.nn.sigmoid(a)) * b
        o = jnp.dot(act, w2_ref[0], preferred_element_type=jnp.float32)
        o_ref[...] = o * wr_ref[...]


_ffn = pl.pallas_call(
    _ffn_body,
    grid_spec=pltpu.PrefetchScalarGridSpec(
        num_scalar_prefetch=2,
        grid=(G,),
        in_specs=[
            pl.BlockSpec((BT, H), lambda g, te, nt: (g, 0)),
            pl.BlockSpec((1, H, 2 * F), lambda g, te, nt: (te[g], 0, 0)),
            pl.BlockSpec((1, F, H), lambda g, te, nt: (te[g], 0, 0)),
            pl.BlockSpec((BT, 1), lambda g, te, nt: (g, 0)),
        ],
        out_specs=pl.BlockSpec((BT, H), lambda g, te, nt: (g, 0)),
    ),
    out_shape=jax.ShapeDtypeStruct((R_PAD, H), jnp.float32),
    compiler_params=pltpu.CompilerParams(dimension_semantics=("arbitrary",)),
)


def kernel(hidden_states, routing_weights, W1, W2, selected_experts):
    i32 = jnp.int32
    e_flat = selected_experts.reshape(-1).astype(i32)
    order = jnp.argsort(e_flat).astype(i32)
    e_sorted = e_flat[order]
    counts = jnp.bincount(e_flat, length=NE).astype(i32)
    tiles_pe = (counts + BT - 1) // BT
    tile_start = jnp.concatenate(
        [jnp.zeros((1,), i32), jnp.cumsum(tiles_pe)[:-1].astype(i32)])
    n_tiles = jnp.sum(tiles_pe).astype(i32)
    row_start = tile_start * BT
    grp_start = jnp.concatenate(
        [jnp.zeros((1,), i32), jnp.cumsum(counts)[:-1].astype(i32)])
    p = jnp.arange(NT * NK, dtype=i32)
    rank = p - grp_start[e_sorted]
    dest_sorted = row_start[e_sorted] + rank
    src_tok = jnp.zeros((R_PAD,), i32).at[dest_sorted].set(order // NK)
    w_pad = jnp.zeros((R_PAD,), jnp.float32).at[dest_sorted].set(
        routing_weights.reshape(-1)[order])
    dest = jnp.zeros((NT * NK,), i32).at[order].set(dest_sorted)
    pos = dest.reshape(NT, NK)
    g_idx = jnp.arange(G, dtype=i32)
    g_eff = jnp.minimum(g_idx, n_tiles - 1)
    tile_expert = (jnp.searchsorted(tile_start, g_eff, side="right")
                   .astype(i32) - 1)
    tile_expert = jnp.clip(tile_expert, 0, NE - 1)

    dispatch, combine = _sc_kernels()
    x_pad = dispatch(src_tok, hidden_states)
    out_pad = _ffn(tile_expert, n_tiles.reshape(1), x_pad, W1, W2,
                   w_pad.reshape(R_PAD, 1))
    final = combine(pos[:, 0], pos[:, 1], out_pad)
    return final


# speedup vs baseline: 4.5825x; 2.7225x over previous
import functools

import jax
import jax.numpy as jnp
from jax import lax
from jax.experimental import pallas as pl
from jax.experimental.pallas import tpu as pltpu
from jax.experimental.pallas import tpu_sc as plsc

H = 768
F = 1024
NE = 64
NT = 2048
NK = 2
BT = 128
G = 96
R_PAD = G * BT

NC, NS = 2, 16
NW = NC * NS
TPW = NT // NW

@functools.lru_cache(maxsize=1)
def _sc_kernels():
    mesh = plsc.VectorSubcoreMesh(core_axis_name="c", subcore_axis_name="s")

    @functools.partial(
        pl.kernel,
        out_type=jax.ShapeDtypeStruct((NT, H), jnp.float32),
        mesh=mesh,
        scratch_types=[
            pltpu.VMEM((TPW,), jnp.int32),
            pltpu.VMEM((TPW,), jnp.int32),
            pltpu.VMEM((TPW, H), jnp.float32),
            pltpu.VMEM((TPW, H), jnp.float32),
            pltpu.SemaphoreType.DMA,
        ],
    )
    def combine(idx0_hbm, idx1_hbm, opad_hbm, fin_hbm, i0_v, i1_v, r0_v, r1_v, sem):
        wid = lax.axis_index("s") * NC + lax.axis_index("c")
        base = wid * TPW
        pltpu.sync_copy(idx0_hbm.at[pl.ds(base, TPW)], i0_v)
        pltpu.sync_copy(idx1_hbm.at[pl.ds(base, TPW)], i1_v)
        cp0 = pltpu.async_copy(opad_hbm.at[i0_v], r0_v, sem)
        cp1 = pltpu.async_copy(opad_hbm.at[i1_v], r1_v, sem)
        cp0.wait()
        cp1.wait()

        def add_row(t, carry):
            for j in range(H // 16):
                sl = pl.ds(j * 16, 16)
                r0_v[t, sl] = r0_v[t, sl] + r1_v[t, sl]
            return carry

        lax.fori_loop(0, TPW, add_row, 0)
        pltpu.sync_copy(r0_v, fin_hbm.at[pl.ds(base, TPW)])

    return combine


def _ffn_body(te_ref, nt_ref, tok_ref, hs_ref, w1_ref, w2_ref, wr_ref, o_ref):
    g = pl.program_id(0)

    @pl.when(g < nt_ref[0])
    def _():
        tok_col = jnp.transpose(tok_ref[0])
        row_t = lax.broadcasted_iota(jnp.int32, (BT, NT), 1)
        p_onehot = jnp.where(row_t == tok_col, 1.0, 0.0)
        x = jnp.dot(p_onehot, hs_ref[...],
                    preferred_element_type=jnp.float32)
        h = jnp.dot(x, w1_ref[0], preferred_element_type=jnp.float32)
        a = h[:, :F]
        b = h[:, F:]
        act = (a * jax.nn.sigmoid(a)) * b
        o = jnp.dot(act, w2_ref[0], preferred_element_type=jnp.float32)
        o_ref[...] = o * wr_ref[...]


def _tix(g, te, nt):
    return jnp.minimum(g, nt[0] - 1)


_ffn = pl.pallas_call(
    _ffn_body,
    grid_spec=pltpu.PrefetchScalarGridSpec(
        num_scalar_prefetch=2,
        grid=(G,),
        in_specs=[
            pl.BlockSpec((1, 1, BT), lambda g, te, nt: (_tix(g, te, nt), 0, 0)),
            pl.BlockSpec((NT, H), lambda g, te, nt: (0, 0)),
            pl.BlockSpec((1, H, 2 * F), lambda g, te, nt: (te[g], 0, 0)),
            pl.BlockSpec((1, F, H), lambda g, te, nt: (te[g], 0, 0)),
            pl.BlockSpec((BT, 1), lambda g, te, nt: (_tix(g, te, nt), 0)),
        ],
        out_specs=pl.BlockSpec((BT, H), lambda g, te, nt: (_tix(g, te, nt), 0)),
    ),
    out_shape=jax.ShapeDtypeStruct((R_PAD, H), jnp.float32),
    compiler_params=pltpu.CompilerParams(dimension_semantics=("arbitrary",)),
)


def kernel(hidden_states, routing_weights, W1, W2, selected_experts):
    i32 = jnp.int32
    e_flat = selected_experts.reshape(-1).astype(i32)
    onehot = (e_flat[:, None] == jnp.arange(NE, dtype=i32)[None, :]).astype(i32)
    run = jnp.cumsum(onehot, axis=0)
    counts = run[-1, :]
    rank = jnp.sum(onehot * run, axis=1) - 1
    tiles_pe = (counts + BT - 1) // BT
    tile_start = jnp.concatenate(
        [jnp.zeros((1,), i32), jnp.cumsum(tiles_pe)[:-1].astype(i32)])
    n_tiles = jnp.sum(tiles_pe).astype(i32)
    row_start = tile_start * BT
    dest = jnp.sum(onehot * row_start[None, :], axis=1) + rank
    tok_and_w = jnp.stack(
        [jnp.arange(NT * NK, dtype=i32) // NK,
         jax.lax.bitcast_convert_type(routing_weights.reshape(-1), i32)],
        axis=1)
    packed = jnp.zeros((R_PAD, 2), i32).at[dest].set(tok_and_w)
    src_tok = packed[:, 0]
    w_pad = jax.lax.bitcast_convert_type(packed[:, 1], jnp.float32)
    pos = dest.reshape(NT, NK)
    g_eff = jnp.minimum(jnp.arange(G, dtype=i32), n_tiles - 1)
    tile_expert = jnp.clip(
        jnp.sum((tile_start[None, :] <= g_eff[:, None]).astype(i32), axis=1) - 1,
        0, NE - 1)

    combine = _sc_kernels()
    out_pad = _ffn(tile_expert, n_tiles.reshape(1), src_tok.reshape(G, 1, BT),
                   hidden_states, W1, W2, w_pad.reshape(R_PAD, 1))
    final = combine(pos[:, 0], pos[:, 1], out_pad)
    return final


# speedup vs baseline: 4.6254x; 1.0093x over previous
import functools

import jax
import jax.numpy as jnp
from jax import lax
from jax.experimental import pallas as pl
from jax.experimental.pallas import tpu as pltpu
from jax.experimental.pallas import tpu_sc as plsc

H = 768
F = 1024
NE = 64
NT = 2048
NK = 2
BT = 128
G = 96
R_PAD = G * BT

NC, NS = 2, 16
NW = NC * NS
TPW = NT // NW

@functools.lru_cache(maxsize=1)
def _sc_kernels():
    mesh = plsc.VectorSubcoreMesh(core_axis_name="c", subcore_axis_name="s")

    HTPW = TPW // 2

    @functools.partial(
        pl.kernel,
        out_type=jax.ShapeDtypeStruct((NT, H), jnp.float32),
        mesh=mesh,
        scratch_types=[
            pltpu.VMEM((TPW,), jnp.int32),
            pltpu.VMEM((TPW,), jnp.int32),
            pltpu.VMEM((HTPW, H), jnp.float32),
            pltpu.VMEM((HTPW, H), jnp.float32),
            pltpu.VMEM((HTPW, H), jnp.float32),
            pltpu.VMEM((HTPW, H), jnp.float32),
            pltpu.SemaphoreType.DMA,
            pltpu.SemaphoreType.DMA,
        ],
    )
    def combine(idx0_hbm, idx1_hbm, opad_hbm, fin_hbm, i0_v, i1_v,
                r0a, r1a, r0b, r1b, gsem, ssem):
        wid = lax.axis_index("s") * NC + lax.axis_index("c")
        base = wid * TPW
        ci0 = pltpu.async_copy(idx0_hbm.at[pl.ds(base, TPW)], i0_v, gsem)
        ci1 = pltpu.async_copy(idx1_hbm.at[pl.ds(base, TPW)], i1_v, gsem)
        ci0.wait()
        ci1.wait()
        cpa0 = pltpu.async_copy(opad_hbm.at[i0_v.at[pl.ds(0, HTPW)]], r0a, gsem)
        cpa1 = pltpu.async_copy(opad_hbm.at[i1_v.at[pl.ds(0, HTPW)]], r1a, gsem)
        cpb0 = pltpu.async_copy(opad_hbm.at[i0_v.at[pl.ds(HTPW, HTPW)]], r0b, gsem)
        cpb1 = pltpu.async_copy(opad_hbm.at[i1_v.at[pl.ds(HTPW, HTPW)]], r1b, gsem)

        def add_rows(dst, src):
            def body(t, carry):
                for j in range(H // 16):
                    sl = pl.ds(j * 16, 16)
                    dst[t, sl] = dst[t, sl] + src[t, sl]
                return carry
            lax.fori_loop(0, HTPW, body, 0)

        cpa0.wait()
        cpa1.wait()
        add_rows(r0a, r1a)
        sta = pltpu.async_copy(r0a, fin_hbm.at[pl.ds(base, HTPW)], ssem)
        cpb0.wait()
        cpb1.wait()
        add_rows(r0b, r1b)
        stb = pltpu.async_copy(r0b, fin_hbm.at[pl.ds(base + HTPW, HTPW)], ssem)
        sta.wait()
        stb.wait()

    return combine


def _ffn_body(te_ref, nt_ref, tok_ref, hs_ref, w1_ref, w2_ref, wr_ref, o_ref):
    g = pl.program_id(0)

    @pl.when(g < nt_ref[0])
    def _():
        tok_col = jnp.transpose(tok_ref[0])
        row_t = lax.broadcasted_iota(jnp.int32, (BT, NT), 1)
        p_onehot = jnp.where(row_t == tok_col, 1.0, 0.0)
        x = jnp.dot(p_onehot, hs_ref[...],
                    preferred_element_type=jnp.float32)
        h = jnp.dot(x, w1_ref[0], preferred_element_type=jnp.float32)
        a = h[:, :F]
        b = h[:, F:]
        act = (a * jax.nn.sigmoid(a)) * b
        o = jnp.dot(act, w2_ref[0], preferred_element_type=jnp.float32)
        o_ref[...] = o * wr_ref[...]


def _tix(g, te, nt):
    return jnp.minimum(g, nt[0] - 1)


_ffn = pl.pallas_call(
    _ffn_body,
    grid_spec=pltpu.PrefetchScalarGridSpec(
        num_scalar_prefetch=2,
        grid=(G,),
        in_specs=[
            pl.BlockSpec((1, 1, BT), lambda g, te, nt: (_tix(g, te, nt), 0, 0)),
            pl.BlockSpec((NT, H), lambda g, te, nt: (0, 0)),
            pl.BlockSpec((1, H, 2 * F), lambda g, te, nt: (te[g], 0, 0)),
            pl.BlockSpec((1, F, H), lambda g, te, nt: (te[g], 0, 0)),
            pl.BlockSpec((BT, 1), lambda g, te, nt: (_tix(g, te, nt), 0)),
        ],
        out_specs=pl.BlockSpec((BT, H), lambda g, te, nt: (_tix(g, te, nt), 0)),
    ),
    out_shape=jax.ShapeDtypeStruct((R_PAD, H), jnp.float32),
    compiler_params=pltpu.CompilerParams(dimension_semantics=("arbitrary",)),
)


def kernel(hidden_states, routing_weights, W1, W2, selected_experts):
    i32 = jnp.int32
    e_flat = selected_experts.reshape(-1).astype(i32)
    onehot = (e_flat[:, None] == jnp.arange(NE, dtype=i32)[None, :]).astype(i32)
    run = jnp.cumsum(onehot, axis=0)
    counts = run[-1, :]
    rank = jnp.sum(onehot * run, axis=1) - 1
    tiles_pe = (counts + BT - 1) // BT
    tile_start = jnp.concatenate(
        [jnp.zeros((1,), i32), jnp.cumsum(tiles_pe)[:-1].astype(i32)])
    n_tiles = jnp.sum(tiles_pe).astype(i32)
    row_start = tile_start * BT
    dest = jnp.sum(onehot * row_start[None, :], axis=1) + rank
    tok_and_w = jnp.stack(
        [jnp.arange(NT * NK, dtype=i32) // NK,
         jax.lax.bitcast_convert_type(routing_weights.reshape(-1), i32)],
        axis=1)
    packed = jnp.zeros((R_PAD, 2), i32).at[dest].set(tok_and_w)
    src_tok = packed[:, 0]
    w_pad = jax.lax.bitcast_convert_type(packed[:, 1], jnp.float32)
    pos = dest.reshape(NT, NK)
    g_eff = jnp.minimum(jnp.arange(G, dtype=i32), n_tiles - 1)
    tile_expert = jnp.clip(
        jnp.sum((tile_start[None, :] <= g_eff[:, None]).astype(i32), axis=1) - 1,
        0, NE - 1)

    combine = _sc_kernels()
    out_pad = _ffn(tile_expert, n_tiles.reshape(1), src_tok.reshape(G, 1, BT),
                   hidden_states, W1, W2, w_pad.reshape(R_PAD, 1))
    final = combine(pos[:, 0], pos[:, 1], out_pad)
    return final
